# baseline (speedup 1.0000x reference)
import functools

import jax
import jax.numpy as jnp
from jax import lax
from jax.experimental import pallas as pl
from jax.experimental.pallas import tpu as pltpu
from jax.experimental.pallas import tpu_sc as plsc

N = 10000
NP = 10240
E = 320000
H = 128
HH = 32
NQ = H // HH
C = 5
NC = 2
NS = 16
EPT = E // NS
CK = 128
NCHUNK = 160
EPT_PAD = NCHUNK * CK
RPT = NP // NS
BR = 256
GRID = NP // BR

_mesh = plsc.VectorSubcoreMesh(
    core_axis_name="c", subcore_axis_name="s", num_cores=NC, num_subcores=NS)
_sc_params = pltpu.CompilerParams(needs_layout_passes=False,
                                  use_tc_tiling_on_sc=False)


@functools.partial(
    pl.kernel,
    out_type=jax.ShapeDtypeStruct((NC, NS, NP), jnp.float32),
    mesh=_mesh,
    scratch_types=[
        pltpu.VMEM((EPT_PAD,), jnp.int32),
        pltpu.VMEM((NP,), jnp.float32),
    ],
    compiler_params=_sc_params,
)
def _deg_kernel(dstf, znp, degp, idx_v, acc_v):
    c = lax.axis_index("c")
    s = lax.axis_index("s")
    pltpu.sync_copy(dstf.at[c, s], idx_v)
    pltpu.sync_copy(znp, acc_v)
    ones16 = jnp.ones((16,), jnp.float32)

    def sc_body(i, carry):
        idx = idx_v[pl.ds(i * 16, 16)]
        plsc.addupdate_scatter(acc_v, [idx], ones16)
        return carry

    lax.fori_loop(0, EPT_PAD // 16, sc_body, 0, unroll=4)
    pltpu.sync_copy(acc_v, degp.at[c, s])


def _make_agg(with_sigma):
    out_types = [jax.ShapeDtypeStruct((NC, NP, HH), jnp.float32)
                 for _ in range(NQ)]
    scratch = [
        pltpu.VMEM((EPT_PAD,), jnp.int32),
        pltpu.VMEM((NCHUNK, CK), jnp.int32),
        pltpu.VMEM((CK, HH), jnp.float32),
        pltpu.VMEM((CK, HH), jnp.float32),
        pltpu.VMEM_SHARED((NP, HH), jnp.float32),
    ]
    if with_sigma:
        out_types.append(jax.ShapeDtypeStruct((NC, NS, NP), jnp.float32))
        scratch += [
            pltpu.VMEM((EPT_PAD,), jnp.int32),
            pltpu.VMEM((NP,), jnp.float32),
            pltpu.VMEM((NP,), jnp.float32),
        ]

    def body(*refs):
        if with_sigma:
            y1q = refs[0:NQ]
            y2q = refs[NQ:2 * NQ]
            (dinvp, srcf, dstc, dstf, zck, znp) = refs[2 * NQ:2 * NQ + 6]
            pq_out = refs[2 * NQ + 6:3 * NQ + 6]
            sig_out = refs[3 * NQ + 6]
            (srcf_v, dstc_v, g0, g1, acc,
             dstf_v, dinv_v, sig_v) = refs[3 * NQ + 7:]
        else:
            y1q = refs[0:NQ]
            y2q = refs[NQ:2 * NQ]
            (srcf, dstc, zck) = refs[2 * NQ:2 * NQ + 3]
            pq_out = refs[2 * NQ + 3:3 * NQ + 3]
            (srcf_v, dstc_v, g0, g1, acc) = refs[3 * NQ + 3:]
        gb = (g0, g1)
        c = lax.axis_index("c")
        s = lax.axis_index("s")
        pltpu.sync_copy(srcf.at[c, s], srcf_v)
        pltpu.sync_copy(dstc.at[c, s], dstc_v)
        NW = NCHUNK // 2

        def half_pass(ya, yb, p_out, gsem):
            pltpu.sync_copy(zck, g0)
            for k in range(RPT // CK):
                pltpu.sync_copy(g0, acc.at[pl.ds(s * RPT + k * CK, CK)])
            plsc.subcore_barrier()

            def edge_loop(y_hbm):
                def start_gather(j, b):
                    pltpu.async_copy(
                        y_hbm.at[srcf_v.at[pl.ds(j * CK, CK)]],
                        gb[b], gsem[b])

                def wait_gather(j, b):
                    pltpu.make_async_copy(
                        y_hbm.at[srcf_v.at[pl.ds(j * CK, CK)]],
                        gb[b], gsem[b]).wait()

                def scatter(j, b):
                    pltpu.sync_copy(gb[b], acc.at[dstc_v.at[j]], add=True)

                start_gather(0, 0)

                def wave(w, carry):
                    j0 = 2 * w
                    j1 = j0 + 1
                    wait_gather(j0, 0)
                    start_gather(j1, 1)
                    scatter(j0, 0)
                    wait_gather(j1, 1)

                    @pl.when(w < NW - 1)
                    def _():
                        start_gather(j1 + 1, 0)

                    scatter(j1, 1)
                    return carry

                lax.fori_loop(0, NW, wave, 0)

            @pl.when(c == 0)
            def _():
                edge_loop(ya)

            @pl.when(c == 1)
            def _():
                edge_loop(yb)

            plsc.subcore_barrier()
            for k in range(RPT // CK):
                off = s * RPT + k * CK
                pltpu.sync_copy(acc.at[pl.ds(off, CK)], g0)
                pltpu.sync_copy(g0, p_out.at[c, pl.ds(off, CK)])
            plsc.subcore_barrier()

        def scoped(gsem):
            for q in range(NQ):
                half_pass(y1q[q], y2q[q], pq_out[q], gsem)

        pl.run_scoped(scoped, (pltpu.SemaphoreType.DMA,
                               pltpu.SemaphoreType.DMA))

        if with_sigma:
            pltpu.sync_copy(dstf.at[c, s], dstf_v)
            pltpu.sync_copy(dinvp.at[c], dinv_v)
            pltpu.sync_copy(znp, sig_v)

            def sig_body(i, carry):
                d16 = dstf_v[pl.ds(i * 16, 16)]
                s16 = srcf_v[pl.ds(i * 16, 16)]
                vals = plsc.load_gather(dinv_v, [d16])
                plsc.addupdate_scatter(sig_v, [s16], vals)
                return carry

            lax.fori_loop(0, EPT_PAD // 16, sig_body, 0, unroll=4)
            pltpu.sync_copy(sig_v, sig_out.at[c, s])

    return pl.kernel(
        body,
        out_type=tuple(out_types),
        mesh=_mesh,
        scratch_types=scratch,
        compiler_params=_sc_params,
    )


_agg_sig = _make_agg(True)
_agg = _make_agg(False)


def _split_q(y):
    return tuple(y[:, q * HH:(q + 1) * HH] for q in range(NQ))


def _tc_b(degp, x1, x2, w1):
    def body(degp_ref, x1_ref, x2_ref, w1_ref, dinv_ref, *yq_refs):
        i = pl.program_id(0)
        deg = jnp.sum(degp_ref[...], axis=1) + 1.0
        rows = i * BR + lax.broadcasted_iota(jnp.int32, (NC, BR), 1)
        dinv = jnp.where(rows < N, lax.rsqrt(deg), 0.0)
        dinv_ref[...] = dinv
        d1 = dinv[0, :][:, None]
        d2 = dinv[1, :][:, None]
        w = w1_ref[...]
        y1 = d1 * jnp.dot(x1_ref[...], w, preferred_element_type=jnp.float32)
        y2 = d2 * jnp.dot(x2_ref[...], w, preferred_element_type=jnp.float32)
        for q in range(NQ):
            yq_refs[q][...] = y1[:, q * HH:(q + 1) * HH]
            yq_refs[NQ + q][...] = y2[:, q * HH:(q + 1) * HH]

    yspec = pl.BlockSpec((BR, HH), lambda i: (i, 0))
    yshape = jax.ShapeDtypeStruct((NP, HH), jnp.float32)
    return pl.pallas_call(
        body,
        grid=(GRID,),
        in_specs=[
            pl.BlockSpec((NC, NS, BR), lambda i: (0, 0, i)),
            pl.BlockSpec((BR, H), lambda i: (i, 0)),
            pl.BlockSpec((BR, H), lambda i: (i, 0)),
            pl.BlockSpec((H, H), lambda i: (0, 0)),
        ],
        out_specs=[pl.BlockSpec((NC, BR), lambda i: (0, i))]
        + [yspec] * (2 * NQ),
        out_shape=[jax.ShapeDtypeStruct((NC, NP), jnp.float32)]
        + [yshape] * (2 * NQ),
    )(degp, x1, x2, w1)


def _tc_d(y1q, y2q, pq, d1, d2, b, w):
    def body(*refs):
        y1r = refs[0:NQ]
        y2r = refs[NQ:2 * NQ]
        pr = refs[2 * NQ:3 * NQ]
        d1_ref, d2_ref, b_ref, w_ref = refs[3 * NQ:3 * NQ + 4]
        o_refs = refs[3 * NQ + 4:]
        d1v = d1_ref[...]
        d2v = d2_ref[...]
        bv = b_ref[...]
        w_ = w_ref[...]
        a1 = jnp.concatenate(
            [y1r[q][...] + pr[q][...][0] for q in range(NQ)], axis=1)
        a2 = jnp.concatenate(
            [y2r[q][...] + pr[q][...][1] for q in range(NQ)], axis=1)
        h1 = jnp.maximum(d1v * a1 + bv, 0.0)
        h2 = jnp.maximum(d2v * a2 + bv, 0.0)
        o1 = d1v * jnp.dot(h1, w_, preferred_element_type=jnp.float32)
        o2 = d2v * jnp.dot(h2, w_, preferred_element_type=jnp.float32)
        for q in range(NQ):
            o_refs[q][...] = o1[:, q * HH:(q + 1) * HH]
            o_refs[NQ + q][...] = o2[:, q * HH:(q + 1) * HH]

    yspec = pl.BlockSpec((BR, HH), lambda i: (i, 0))
    pspec = pl.BlockSpec((NC, BR, HH), lambda i: (0, i, 0))
    yshape = jax.ShapeDtypeStruct((NP, HH), jnp.float32)
    return pl.pallas_call(
        body,
        grid=(GRID,),
        in_specs=[yspec] * (2 * NQ) + [pspec] * NQ + [
            pl.BlockSpec((BR, 1), lambda i: (i, 0)),
            pl.BlockSpec((BR, 1), lambda i: (i, 0)),
            pl.BlockSpec((1, H), lambda i: (0, 0)),
            pl.BlockSpec((H, H), lambda i: (0, 0)),
        ],
        out_specs=[yspec] * (2 * NQ),
        out_shape=[yshape] * (2 * NQ),
    )(*y1q, *y2q, *pq, d1, d2, b, w)


def _tc_f(z1q, z2q, pq, d1, d2, sigp, b):
    def body(*refs):
        y1r = refs[0:NQ]
        y2r = refs[NQ:2 * NQ]
        pr = refs[2 * NQ:3 * NQ]
        d1_ref, d2_ref, sigp_ref, b_ref, s_ref = refs[3 * NQ:]
        i = pl.program_id(0)
        d1v = d1_ref[...]
        d2v = d2_ref[...]
        bv = b_ref[...]
        a1 = jnp.concatenate(
            [y1r[q][...] + pr[q][...][0] for q in range(NQ)], axis=1)
        a2 = jnp.concatenate(
            [y2r[q][...] + pr[q][...][1] for q in range(NQ)], axis=1)
        h1 = jnp.maximum(d1v * a1 + bv, 0.0)
        h2 = jnp.maximum(d2v * a2 + bv, 0.0)
        sig = jnp.sum(sigp_ref[...], axis=1)
        c1 = d1v * sig[0, :][:, None] + d1v * d1v
        c2 = d2v * sig[1, :][:, None] + d2v * d2v
        s1 = jnp.sum(c1 * h1, axis=0)
        s2 = jnp.sum(c2 * h2, axis=0)

        @pl.when(i == 0)
        def _():
            s_ref[...] = jnp.zeros((NC, H), jnp.float32)

        s_ref[...] += jnp.stack([s1, s2])

    yspec = pl.BlockSpec((BR, HH), lambda i: (i, 0))
    pspec = pl.BlockSpec((NC, BR, HH), lambda i: (0, i, 0))
    return pl.pallas_call(
        body,
        grid=(GRID,),
        in_specs=[yspec] * (2 * NQ) + [pspec] * NQ + [
            pl.BlockSpec((BR, 1), lambda i: (i, 0)),
            pl.BlockSpec((BR, 1), lambda i: (i, 0)),
            pl.BlockSpec((NC, NS, BR), lambda i: (0, 0, i)),
            pl.BlockSpec((1, H), lambda i: (0, 0)),
        ],
        out_specs=pl.BlockSpec((NC, H), lambda i: (0, 0)),
        out_shape=jax.ShapeDtypeStruct((NC, H), jnp.float32),
    )(*z1q, *z2q, *pq, d1, d2, sigp, b)


def _tc_g(s_in, w3, b3, wl, bl):
    def body(s_ref, w3_ref, b3_ref, wl_ref, bl_ref, o_ref):
        sv = s_ref[...]
        pooled = jnp.dot((sv[0:1, :] + sv[1:2, :]) * (1.0 / (2 * N)),
                         w3_ref[...], preferred_element_type=jnp.float32)
        pooled = pooled + b3_ref[...]
        o_ref[...] = jnp.dot(pooled, wl_ref[...],
                             preferred_element_type=jnp.float32) + bl_ref[...]

    return pl.pallas_call(
        body,
        out_shape=jax.ShapeDtypeStruct((1, H), jnp.float32),
    )(s_in, w3, b3, wl, bl)


def kernel(x_1, edge_index_1, x_2, edge_index_2,
           W1, b1, W2, b2, W3, b3, Wl, bl):
    x1p = jnp.pad(x_1, ((0, NP - N), (0, 0)))
    x2p = jnp.pad(x_2, ((0, NP - N), (0, 0)))

    def prep(ei):
        a = ei.reshape(2, NS, EPT)
        a = jnp.pad(a, ((0, 0), (0, 0), (0, EPT_PAD - EPT)),
                    constant_values=N)
        return a[0], a[1], a[1].reshape(NS, NCHUNK, CK)

    s1f, d1f, d1c = prep(edge_index_1)
    s2f, d2f, d2c = prep(edge_index_2)
    srcf = jnp.stack([s1f, s2f])
    dstf = jnp.stack([d1f, d2f])
    dstc = jnp.stack([d1c, d2c])
    znp = jnp.zeros((NP,), jnp.float32)
    zck = jnp.zeros((CK, HH), jnp.float32)

    degp = _deg_kernel(dstf, znp)
    bout = _tc_b(degp, x1p, x2p, W1)
    dinvp = bout[0]
    y1q = bout[1:1 + NQ]
    y2q = bout[1 + NQ:1 + 2 * NQ]
    aout = _agg_sig(*y1q, *y2q, dinvp, srcf, dstc, dstf, zck, znp)
    p1q = aout[0:NQ]
    sigp = aout[NQ]
    d1 = dinvp[0][:, None]
    d2 = dinvp[1][:, None]
    zout = _tc_d(y1q, y2q, p1q, d1, d2, b1.reshape(1, H), W2)
    z1q = zout[0:NQ]
    z2q = zout[NQ:2 * NQ]
    p2q = _agg(*z1q, *z2q, srcf, dstc, zck)
    S = _tc_f(z1q, z2q, p2q, d1, d2, sigp, b2.reshape(1, H))
    wlp = jnp.pad(Wl, ((0, 0), (0, H - C)))
    blp = jnp.pad(bl, (0, H - C)).reshape(1, H)
    o = _tc_g(S, W3, b3.reshape(1, H), wlp, blp)
    return o[:, :C]


# speedup vs baseline: 2.3820x; 2.3820x over previous
import functools

import jax
import jax.numpy as jnp
from jax import lax
from jax.experimental import pallas as pl
from jax.experimental.pallas import tpu as pltpu
from jax.experimental.pallas import tpu_sc as plsc

N = 10000
NP = 10240
E = 320000
H = 128
HH = H // 2
C = 5
NC = 2
NS = 16
EPT = E // NS
CK = 128
NCHUNK = 158
EPT_PAD = NCHUNK * CK
RPT = NP // NS
BR = 256
GRID = NP // BR

_mesh = plsc.VectorSubcoreMesh(
    core_axis_name="c", subcore_axis_name="s", num_cores=NC, num_subcores=NS)
_sc_params = pltpu.CompilerParams(needs_layout_passes=False,
                                  use_tc_tiling_on_sc=False)


@functools.partial(
    pl.kernel,
    out_type=jax.ShapeDtypeStruct((NC, NS, NP), jnp.float32),
    mesh=_mesh,
    scratch_types=[
        pltpu.VMEM((EPT_PAD,), jnp.int32),
        pltpu.VMEM((NP,), jnp.float32),
    ],
    compiler_params=_sc_params,
)
def _deg_kernel(dstf, znp, degp, idx_v, acc_v):
    c = lax.axis_index("c")
    s = lax.axis_index("s")
    pltpu.sync_copy(dstf.at[c, s], idx_v)
    pltpu.sync_copy(znp, acc_v)
    ones16 = jnp.ones((16,), jnp.float32)

    def sc_body(i, carry):
        idx = idx_v[pl.ds(i * 16, 16)]
        plsc.addupdate_scatter(acc_v, [idx], ones16)
        return carry

    lax.fori_loop(0, EPT_PAD // 16, sc_body, 0, unroll=4)
    pltpu.sync_copy(acc_v, degp.at[c, s])


def _make_agg(with_sigma):
    out_types = [jax.ShapeDtypeStruct((NC, NP, H), jnp.bfloat16)]
    scratch = [
        pltpu.VMEM((EPT_PAD,), jnp.int32),
        pltpu.VMEM((NCHUNK, CK), jnp.int32),
        pltpu.VMEM((CK, H), jnp.bfloat16),
        pltpu.VMEM_SHARED((NP, H), jnp.bfloat16),
    ]
    if with_sigma:
        out_types.append(jax.ShapeDtypeStruct((NC, NS, NP), jnp.float32))
        scratch += [
            pltpu.VMEM((EPT_PAD,), jnp.int32),
            pltpu.VMEM((NP,), jnp.float32),
            pltpu.VMEM((NP,), jnp.float32),
        ]

    def body(*refs):
        if with_sigma:
            (y1, y2, dinvp, srcf, dstc, dstf, zck, znp,
             p_out, sig_out,
             srcf_v, dstc_v, g0, acc, dstf_v, dinv_v, sig_v) = refs
        else:
            (y1, y2, srcf, dstc, zck,
             p_out,
             srcf_v, dstc_v, g0, acc) = refs
        c = lax.axis_index("c")
        s = lax.axis_index("s")
        pltpu.sync_copy(srcf.at[c, s], srcf_v)
        pltpu.sync_copy(dstc.at[c, s], dstc_v)

        def half_pass(ya, yb, p_out):
            pltpu.sync_copy(zck, g0)
            for k in range(RPT // CK):
                pltpu.sync_copy(g0, acc.at[pl.ds(s * RPT + k * CK, CK)])
            plsc.subcore_barrier()

            def edge_loop(y_hbm):
                def body_j(j, carry):
                    pltpu.sync_copy(y_hbm.at[srcf_v.at[pl.ds(j * CK, CK)]],
                                    g0)
                    pltpu.sync_copy(g0, acc.at[dstc_v.at[j]], add=True)
                    return carry
                lax.fori_loop(0, NCHUNK, body_j, 0)

            @pl.when(c == 0)
            def _():
                edge_loop(ya)

            @pl.when(c == 1)
            def _():
                edge_loop(yb)

            plsc.subcore_barrier()
            for k in range(RPT // CK):
                off = s * RPT + k * CK
                pltpu.sync_copy(acc.at[pl.ds(off, CK)], g0)
                pltpu.sync_copy(g0, p_out.at[c, pl.ds(off, CK)])
            plsc.subcore_barrier()

        half_pass(y1, y2, p_out)

        if with_sigma:
            pltpu.sync_copy(dstf.at[c, s], dstf_v)
            pltpu.sync_copy(dinvp.at[c], dinv_v)
            pltpu.sync_copy(znp, sig_v)

            def sig_body(i, carry):
                d16 = dstf_v[pl.ds(i * 16, 16)]
                s16 = srcf_v[pl.ds(i * 16, 16)]
                vals = plsc.load_gather(dinv_v, [d16])
                plsc.addupdate_scatter(sig_v, [s16], vals)
                return carry

            lax.fori_loop(0, EPT_PAD // 16, sig_body, 0, unroll=4)
            pltpu.sync_copy(sig_v, sig_out.at[c, s])

    return pl.kernel(
        body,
        out_type=tuple(out_types),
        mesh=_mesh,
        scratch_types=scratch,
        compiler_params=_sc_params,
    )


_agg_sig = _make_agg(True)
_agg = _make_agg(False)


def _halves(y):
    return y[:, :HH], y[:, HH:]


def _tc_b(degp, x1, x2, w1):
    def body(degp_ref, x1_ref, x2_ref, w1_ref,
             dinv_ref, y1_ref, y2_ref):
        i = pl.program_id(0)
        deg = jnp.sum(degp_ref[...], axis=1) + 1.0
        rows = i * BR + lax.broadcasted_iota(jnp.int32, (NC, BR), 1)
        dinv = jnp.where(rows < N, lax.rsqrt(deg), 0.0)
        dinv_ref[...] = dinv
        d1 = dinv[0, :][:, None]
        d2 = dinv[1, :][:, None]
        w = w1_ref[...]
        y1 = d1 * jnp.dot(x1_ref[...], w, preferred_element_type=jnp.float32)
        y2 = d2 * jnp.dot(x2_ref[...], w, preferred_element_type=jnp.float32)
        y1_ref[...] = y1.astype(jnp.bfloat16)
        y2_ref[...] = y2.astype(jnp.bfloat16)

    yspec = pl.BlockSpec((BR, H), lambda i: (i, 0))
    yshape = jax.ShapeDtypeStruct((NP, H), jnp.bfloat16)
    return pl.pallas_call(
        body,
        grid=(GRID,),
        in_specs=[
            pl.BlockSpec((NC, NS, BR), lambda i: (0, 0, i)),
            pl.BlockSpec((BR, H), lambda i: (i, 0)),
            pl.BlockSpec((BR, H), lambda i: (i, 0)),
            pl.BlockSpec((H, H), lambda i: (0, 0)),
        ],
        out_specs=[
            pl.BlockSpec((NC, BR), lambda i: (0, i)),
            yspec, yspec,
        ],
        out_shape=[
            jax.ShapeDtypeStruct((NC, NP), jnp.float32),
            yshape, yshape,
        ],
    )(degp, x1, x2, w1)


def _tc_d(y1, y2, p1, d1, d2, b, w):
    def body(y1_ref, y2_ref, p_ref, d1_ref, d2_ref, b_ref, w_ref,
             o1_ref, o2_ref):
        pv = p_ref[...].astype(jnp.float32)
        d1v = d1_ref[...]
        d2v = d2_ref[...]
        bv = b_ref[...]
        w_ = w_ref[...]
        a1 = y1_ref[...].astype(jnp.float32) + pv[0]
        a2 = y2_ref[...].astype(jnp.float32) + pv[1]
        h1 = jnp.maximum(d1v * a1 + bv, 0.0)
        h2 = jnp.maximum(d2v * a2 + bv, 0.0)
        o1 = d1v * jnp.dot(h1, w_, preferred_element_type=jnp.float32)
        o2 = d2v * jnp.dot(h2, w_, preferred_element_type=jnp.float32)
        o1_ref[...] = o1.astype(jnp.bfloat16)
        o2_ref[...] = o2.astype(jnp.bfloat16)

    yspec = pl.BlockSpec((BR, H), lambda i: (i, 0))
    yshape = jax.ShapeDtypeStruct((NP, H), jnp.bfloat16)
    return pl.pallas_call(
        body,
        grid=(GRID,),
        in_specs=[
            yspec, yspec,
            pl.BlockSpec((NC, BR, H), lambda i: (0, i, 0)),
            pl.BlockSpec((BR, 1), lambda i: (i, 0)),
            pl.BlockSpec((BR, 1), lambda i: (i, 0)),
            pl.BlockSpec((1, H), lambda i: (0, 0)),
            pl.BlockSpec((H, H), lambda i: (0, 0)),
        ],
        out_specs=[yspec, yspec],
        out_shape=[yshape, yshape],
    )(y1, y2, p1, d1, d2, b, w)


def _tc_f(y1, y2, p2, d1, d2, sigp, b):
    def body(y1_ref, y2_ref, p_ref, d1_ref, d2_ref, sigp_ref, b_ref, s_ref):
        i = pl.program_id(0)
        pv = p_ref[...].astype(jnp.float32)
        d1v = d1_ref[...]
        d2v = d2_ref[...]
        bv = b_ref[...]
        a1 = y1_ref[...].astype(jnp.float32) + pv[0]
        a2 = y2_ref[...].astype(jnp.float32) + pv[1]
        h1 = jnp.maximum(d1v * a1 + bv, 0.0)
        h2 = jnp.maximum(d2v * a2 + bv, 0.0)
        sig = jnp.sum(sigp_ref[...], axis=1)
        c1 = d1v * sig[0, :][:, None] + d1v * d1v
        c2 = d2v * sig[1, :][:, None] + d2v * d2v
        s1 = jnp.sum(c1 * h1, axis=0)
        s2 = jnp.sum(c2 * h2, axis=0)

        @pl.when(i == 0)
        def _():
            s_ref[...] = jnp.zeros((NC, H), jnp.float32)

        s_ref[...] += jnp.stack([s1, s2])

    yspec = pl.BlockSpec((BR, H), lambda i: (i, 0))
    return pl.pallas_call(
        body,
        grid=(GRID,),
        in_specs=[
            yspec, yspec,
            pl.BlockSpec((NC, BR, H), lambda i: (0, i, 0)),
            pl.BlockSpec((BR, 1), lambda i: (i, 0)),
            pl.BlockSpec((BR, 1), lambda i: (i, 0)),
            pl.BlockSpec((NC, NS, BR), lambda i: (0, 0, i)),
            pl.BlockSpec((1, H), lambda i: (0, 0)),
        ],
        out_specs=pl.BlockSpec((NC, H), lambda i: (0, 0)),
        out_shape=jax.ShapeDtypeStruct((NC, H), jnp.float32),
    )(y1, y2, p2, d1, d2, sigp, b)


def _tc_g(s_in, w3, b3, wl, bl):
    def body(s_ref, w3_ref, b3_ref, wl_ref, bl_ref, o_ref):
        sv = s_ref[...]
        pooled = jnp.dot((sv[0:1, :] + sv[1:2, :]) * (1.0 / (2 * N)),
                         w3_ref[...], preferred_element_type=jnp.float32)
        pooled = pooled + b3_ref[...]
        o_ref[...] = jnp.dot(pooled, wl_ref[...],
                             preferred_element_type=jnp.float32) + bl_ref[...]

    return pl.pallas_call(
        body,
        out_shape=jax.ShapeDtypeStruct((1, H), jnp.float32),
    )(s_in, w3, b3, wl, bl)


def kernel(x_1, edge_index_1, x_2, edge_index_2,
           W1, b1, W2, b2, W3, b3, Wl, bl):
    x1p = jnp.pad(x_1, ((0, NP - N), (0, 0)))
    x2p = jnp.pad(x_2, ((0, NP - N), (0, 0)))

    def prep(ei):
        a = ei.reshape(2, NS, EPT)
        a = jnp.pad(a, ((0, 0), (0, 0), (0, EPT_PAD - EPT)),
                    constant_values=N)
        return a[0], a[1], a[1].reshape(NS, NCHUNK, CK)

    s1f, d1f, d1c = prep(edge_index_1)
    s2f, d2f, d2c = prep(edge_index_2)
    srcf = jnp.stack([s1f, s2f])
    dstf = jnp.stack([d1f, d2f])
    dstc = jnp.stack([d1c, d2c])
    znp = jnp.zeros((NP,), jnp.float32)
    zck = jnp.zeros((CK, H), jnp.bfloat16)

    degp = _deg_kernel(dstf, znp)
    dinvp, y1, y2 = _tc_b(degp, x1p, x2p, W1)
    p1, sigp = _agg_sig(y1, y2, dinvp, srcf, dstc, dstf, zck, znp)
    d1 = dinvp[0][:, None]
    d2 = dinvp[1][:, None]
    z1, z2 = _tc_d(y1, y2, p1, d1, d2, b1.reshape(1, H), W2)
    (p2,) = _agg(z1, z2, srcf, dstc, zck)
    S = _tc_f(z1, z2, p2, d1, d2, sigp, b2.reshape(1, H))
    wlp = jnp.pad(Wl, ((0, 0), (0, H - C)))
    blp = jnp.pad(bl, (0, H - C)).reshape(1, H)
    o = _tc_g(S, W3, b3.reshape(1, H), wlp, blp)
    return o[:, :C]
